# baseline (speedup 1.0000x reference)
import dataclasses
import functools

import jax
import jax.numpy as jnp
from jax import lax
from jax.experimental import pallas as pl
from jax.experimental.pallas import tpu as pltpu
from jax.experimental.pallas import tpu_sc as plsc

NUM_SEG = 64
D = 512
N = 50000
LANES = 16
NC = 2
NS = 16
NW = NC * NS
NVREG = D // LANES

B = 25
NBLK = N // B
BPT = 64
WIN0MAX = NBLK - BPT
XWIN = 32

R_TC = 2000
GRID_TC = N // R_TC
BLK_TC = R_TC // B

_mesh = plsc.VectorSubcoreMesh(core_axis_name="c", subcore_axis_name="s")

_sc_params = pltpu.CompilerParams()
if "needs_layout_passes" in pltpu.CompilerParams.__dataclass_fields__:
    _sc_params = dataclasses.replace(_sc_params, needs_layout_passes=False)


def _tc_block_body(x_ref, p_ref):
    xr = x_ref[...]
    p_ref[...] = jnp.sum(xr.reshape(BLK_TC, B, D), axis=1)


def _tc_block_sums(x):
    return pl.pallas_call(
        _tc_block_body,
        grid=(GRID_TC,),
        in_specs=[pl.BlockSpec((R_TC, D), lambda i: (i, 0))],
        out_specs=pl.BlockSpec((BLK_TC, D), lambda i: (i, 0)),
        out_shape=jax.ShapeDtypeStruct((NBLK, D), jnp.float32),
        compiler_params=pltpu.CompilerParams(
            dimension_semantics=("parallel",)),
    )(x)


@functools.partial(
    pl.kernel,
    mesh=_mesh,
    compiler_params=_sc_params,
    out_type=(
        jax.ShapeDtypeStruct((NW, NUM_SEG, D), jnp.float32),
        jax.ShapeDtypeStruct((NW, NUM_SEG, LANES), jnp.float32),
    ),
    scratch_types=[
        pltpu.VMEM((BPT, D), jnp.float32),
        pltpu.VMEM((BPT * B,), jnp.int32),
        pltpu.VMEM((XWIN, D), jnp.float32),
        pltpu.VMEM((NUM_SEG, D), jnp.float32),
        pltpu.VMEM((NUM_SEG, LANES), jnp.float32),
        pltpu.SemaphoreType.DMA,
        pltpu.SemaphoreType.DMA,
        pltpu.SemaphoreType.DMA,
        pltpu.SemaphoreType.DMA,
    ],
)
def _sc_seg_sum(p_hbm, b_hbm, x_hbm, psum_hbm, pcnt_hbm,
                pchunk, bchunk, xrow, acc, cnt, semp, semb, semx, semo):
    wid = lax.axis_index("s") * NC + lax.axis_index("c")
    blk0 = wid * BPT
    nblk = jnp.maximum(0, jnp.minimum(BPT, NBLK - blk0))
    win0 = jnp.minimum(blk0, WIN0MAX)
    loc0 = blk0 - win0

    zeros16 = jnp.zeros((LANES,), jnp.float32)
    ones16 = jnp.ones((LANES,), jnp.float32)
    bsize16 = jnp.full((LANES,), float(B), jnp.float32)

    pltpu.make_async_copy(
        p_hbm.at[pl.ds(win0, BPT)], pchunk, semp).start()
    pltpu.make_async_copy(
        b_hbm.at[pl.ds(win0 * B, BPT * B)], bchunk, semb).start()

    @pl.loop(0, NUM_SEG)
    def _zero(r):
        for j in range(NVREG):
            acc[r, pl.ds(j * LANES, LANES)] = zeros16
        cnt[r, pl.ds(0, LANES)] = zeros16

    pltpu.make_async_copy(
        p_hbm.at[pl.ds(win0, BPT)], pchunk, semp).wait()
    pltpu.make_async_copy(
        b_hbm.at[pl.ds(win0 * B, BPT * B)], bchunk, semb).wait()

    @pl.loop(0, nblk)
    def _blocks(bi):
        li = loc0 + bi
        sv_a = bchunk[pl.ds(li * B, LANES)]
        sv_b = bchunk[pl.ds(li * B + B - LANES, LANES)]
        sfirst = sv_a[0]
        slast = sv_b[LANES - 1]

        @pl.when(sfirst == slast)
        def _uniform():
            for j in range(NVREG):
                sl = pl.ds(j * LANES, LANES)
                plsc.addupdate(acc.at[sfirst, sl], pchunk[li, sl])
            plsc.addupdate(cnt.at[sfirst, pl.ds(0, LANES)], bsize16)

        @pl.when(sfirst != slast)
        def _boundary():
            g = blk0 + bi
            r0 = g * B
            a0 = pl.multiple_of((r0 // 8) * 8, 8)
            off = r0 - a0
            pltpu.make_async_copy(
                x_hbm.at[pl.ds(a0, XWIN)], xrow, semx).start()
            pltpu.make_async_copy(
                x_hbm.at[pl.ds(a0, XWIN)], xrow, semx).wait()
            for k in range(LANES):
                s = sv_a[k]
                for j in range(NVREG):
                    sl = pl.ds(j * LANES, LANES)
                    plsc.addupdate(acc.at[s, sl], xrow[off + k, sl])
                plsc.addupdate(cnt.at[s, pl.ds(0, LANES)], ones16)
            for k in range(B - LANES):
                s = sv_b[k + 2 * LANES - B]
                for j in range(NVREG):
                    sl = pl.ds(j * LANES, LANES)
                    plsc.addupdate(
                        acc.at[s, sl], xrow[off + LANES + k, sl])
                plsc.addupdate(cnt.at[s, pl.ds(0, LANES)], ones16)

    pltpu.make_async_copy(acc, psum_hbm.at[wid], semo).start()
    pltpu.make_async_copy(cnt, pcnt_hbm.at[wid], semo).start()
    pltpu.make_async_copy(acc, psum_hbm.at[wid], semo).wait()
    pltpu.make_async_copy(cnt, pcnt_hbm.at[wid], semo).wait()


def _combine_body(ps_ref, pc_ref, out_ref):
    sums = jnp.sum(ps_ref[...], axis=0)
    counts = jnp.sum(pc_ref[...], axis=0)[:, 0:1]
    out_ref[...] = sums / jnp.clip(counts, 1.0, None)


def _tc_combine(psum, pcnt):
    return pl.pallas_call(
        _combine_body,
        out_shape=jax.ShapeDtypeStruct((NUM_SEG, D), jnp.float32),
    )(psum, pcnt)


@jax.jit
def kernel(x, batch):
    batch32 = batch.astype(jnp.int32)
    p = _tc_block_sums(x)
    psum, pcnt = _sc_seg_sum(p, batch32, x)
    return _tc_combine(psum, pcnt)


# speedup vs baseline: 1.6021x; 1.6021x over previous
import dataclasses
import functools

import jax
import jax.numpy as jnp
from jax import lax
from jax.experimental import pallas as pl
from jax.experimental.pallas import tpu as pltpu
from jax.experimental.pallas import tpu_sc as plsc

NUM_SEG = 64
D = 512
N = 50000
LANES = 16
NC = 2
NS = 16
NW = NC * NS
NVREG = D // LANES

N_SC = 20000
SUP = 80
NSUP = N_SC // SUP
BPT = -(-NSUP // NW)
TAIL = NSUP - (NW - 1) * BPT

R_TC = 1000
SKIP_TC = N_SC // R_TC
GRID_TC = (N - N_SC) // R_TC

_mesh = plsc.VectorSubcoreMesh(core_axis_name="c", subcore_axis_name="s")

_sc_params = pltpu.CompilerParams()
if "needs_layout_passes" in pltpu.CompilerParams.__dataclass_fields__:
    _sc_params = dataclasses.replace(_sc_params, needs_layout_passes=False)


@functools.partial(
    pl.kernel,
    mesh=_mesh,
    compiler_params=_sc_params,
    out_type=(
        jax.ShapeDtypeStruct((NW, NUM_SEG, D), jnp.float32),
        jax.ShapeDtypeStruct((NW, NUM_SEG, LANES), jnp.float32),
    ),
    scratch_types=[
        pltpu.VMEM((2 * SUP, D), jnp.float32),
        pltpu.VMEM((2 * SUP,), jnp.int32),
        pltpu.VMEM((NUM_SEG, D), jnp.float32),
        pltpu.VMEM((NUM_SEG, LANES), jnp.float32),
        pltpu.SemaphoreType.DMA,
        pltpu.SemaphoreType.DMA,
        pltpu.SemaphoreType.DMA,
    ],
)
def _sc_seg_sum(x_hbm, b_hbm, psum_hbm, pcnt_hbm,
                xbuf, bbuf, acc, cnt, semx, semb, semo):
    cid = lax.axis_index("c")
    sid = lax.axis_index("s")
    wid = sid * NC + cid
    blk0 = wid * BPT
    nblocks = jnp.where(wid == NW - 1, TAIL, BPT)

    zeros16 = jnp.zeros((LANES,), jnp.float32)
    ones16 = jnp.ones((LANES,), jnp.float32)
    sixteen16 = jnp.full((LANES,), 16.0, jnp.float32)

    @pl.loop(0, NUM_SEG)
    def _zero(r):
        for j in range(NVREG):
            acc[r, pl.ds(j * LANES, LANES)] = zeros16
        cnt[r, pl.ds(0, LANES)] = zeros16

    def x_copy(i, base):
        return pltpu.make_async_copy(
            x_hbm.at[pl.ds((blk0 + i) * SUP, SUP)],
            xbuf.at[pl.ds(base, SUP)], semx)

    def b_copy(i, base):
        return pltpu.make_async_copy(
            b_hbm.at[pl.ds((blk0 + i) * SUP, SUP)],
            bbuf.at[pl.ds(base, SUP)], semb)

    x_copy(0, 0).start()
    b_copy(0, 0).start()
    x_copy(1, SUP).start()
    b_copy(1, SUP).start()

    @pl.loop(0, nblocks)
    def _blocks(p):
        base = (p % 2) * SUP
        x_copy(0, 0).wait()
        b_copy(0, 0).wait()

        @pl.loop(0, SUP, step=LANES)
        def _group(goff):
            r0 = base + goff
            svec = bbuf[pl.ds(r0, LANES)]
            sfirst = svec[0]
            slast = svec[LANES - 1]

            @pl.when(sfirst == slast)
            def _uniform():
                def tree(vals):
                    while len(vals) > 1:
                        nxt = [vals[i] + vals[i + 1]
                               for i in range(0, len(vals) - 1, 2)]
                        if len(vals) % 2:
                            nxt.append(vals[-1])
                        vals = nxt
                    return vals[0]

                for j in range(NVREG):
                    sl = pl.ds(j * LANES, LANES)
                    plsc.addupdate(
                        acc.at[sfirst, sl],
                        tree([xbuf[r0 + k, sl] for k in range(LANES)]))
                plsc.addupdate(cnt.at[sfirst, pl.ds(0, LANES)], sixteen16)

            @pl.when(sfirst != slast)
            def _boundary():
                for k in range(LANES):
                    s = svec[k]
                    for j in range(NVREG):
                        sl = pl.ds(j * LANES, LANES)
                        plsc.addupdate(acc.at[s, sl], xbuf[r0 + k, sl])
                    plsc.addupdate(cnt.at[s, pl.ds(0, LANES)], ones16)

        @pl.when(p + 2 < nblocks)
        def _prefetch():
            x_copy(p + 2, base).start()
            b_copy(p + 2, base).start()

    pltpu.make_async_copy(acc, psum_hbm.at[wid], semo).start()
    pltpu.make_async_copy(cnt, pcnt_hbm.at[wid], semo).start()
    pltpu.make_async_copy(acc, psum_hbm.at[wid], semo).wait()
    pltpu.make_async_copy(cnt, pcnt_hbm.at[wid], semo).wait()


def _tc_body(batch_ref, x_ref, sum_ref, cnt_ref):
    i = pl.program_id(0)

    @pl.when(i == 0)
    def _init():
        sum_ref[...] = jnp.zeros_like(sum_ref)
        cnt_ref[...] = jnp.zeros_like(cnt_ref)

    b = batch_ref[0, 0, :]
    onehot = (b[:, None] == jax.lax.broadcasted_iota(
        jnp.int32, (R_TC, NUM_SEG), 1)).astype(jnp.bfloat16)
    xb = x_ref[...].astype(jnp.bfloat16)
    sum_ref[...] += jax.lax.dot_general(
        onehot, xb, (((0,), (0,)), ((), ())),
        preferred_element_type=jnp.float32)
    cnt_ref[...] += jax.lax.dot_general(
        onehot, jnp.ones((R_TC, 128), jnp.bfloat16),
        (((0,), (0,)), ((), ())),
        preferred_element_type=jnp.float32)


def _tc_partial(x, batch3):
    return pl.pallas_call(
        _tc_body,
        grid=(GRID_TC,),
        in_specs=[
            pl.BlockSpec((1, 1, R_TC), lambda i: (i + SKIP_TC, 0, 0)),
            pl.BlockSpec((R_TC, D), lambda i: (i + SKIP_TC, 0)),
        ],
        out_specs=[
            pl.BlockSpec((NUM_SEG, D), lambda i: (0, 0)),
            pl.BlockSpec((NUM_SEG, 128), lambda i: (0, 0)),
        ],
        out_shape=[
            jax.ShapeDtypeStruct((NUM_SEG, D), jnp.float32),
            jax.ShapeDtypeStruct((NUM_SEG, 128), jnp.float32),
        ],
        compiler_params=pltpu.CompilerParams(
            dimension_semantics=("arbitrary",)),
    )(batch3, x)


def _combine_body(ps_ref, pc_ref, ts_ref, tcnt_ref, out_ref):
    sums = jnp.sum(ps_ref[...], axis=0) + ts_ref[...]
    counts = jnp.sum(pc_ref[...], axis=0)[:, 0:1] + tcnt_ref[:, 0:1]
    out_ref[...] = sums / jnp.clip(counts, 1.0, None)


def _tc_combine(psum, pcnt, tsum, tcnt):
    return pl.pallas_call(
        _combine_body,
        out_shape=jax.ShapeDtypeStruct((NUM_SEG, D), jnp.float32),
    )(psum, pcnt, tsum, tcnt)


@jax.jit
def kernel(x, batch):
    batch32 = batch.astype(jnp.int32)
    batch3 = batch32.reshape(N // R_TC, 1, R_TC)
    psum, pcnt = _sc_seg_sum(x, batch32)
    tsum, tcnt = _tc_partial(x, batch3)
    return _tc_combine(psum, pcnt, tsum, tcnt)
